# Initial kernel scaffold; baseline (speedup 1.0000x reference)
import jax
import jax.numpy as jnp
from jax.experimental import pallas as pl


def kernel(x, rij, vij, edge_index, W1, b1, W2, b2):
    raise NotImplementedError("write your pallas kernel here")


# speedup vs baseline: 59.2911x; 59.2911x over previous
import functools

import jax
import jax.numpy as jnp
from jax import lax
from jax.experimental import pallas as pl
from jax.experimental.pallas import tpu as pltpu
from jax.experimental.pallas import tpu_sc as plsc

N = 10000
E = 320000
ATOM_F = 128
DIP_F = 64

_NS = 16
_EPT = 20480
_EPAD = _NS * _EPT
_B = 512
_NB = _B // 128
_NCHUNK = _EPT // _B
_ROWS_PT = N // _NS


def _mlp_body(x_ref, w1_ref, b1_ref, w2_ref, b2_ref, out_ref):
    h = jnp.dot(x_ref[...], w1_ref[...], preferred_element_type=jnp.float32)
    h = h + b1_ref[...]
    h = h * jax.nn.sigmoid(h)
    q = jnp.dot(h, w2_ref[...], preferred_element_type=jnp.float32)
    q = q + b2_ref[...]
    q = q * jax.nn.sigmoid(q)
    out_ref[0] = q[:, :32]
    out_ref[1] = q[:, 32:]


def _mlp(x, W1, b1, W2, b2):
    R = 1000
    grid = (N // R,)
    return pl.pallas_call(
        _mlp_body,
        grid=grid,
        in_specs=[
            pl.BlockSpec((R, ATOM_F), lambda i: (i, 0)),
            pl.BlockSpec((ATOM_F, ATOM_F), lambda i: (0, 0)),
            pl.BlockSpec((1, ATOM_F), lambda i: (0, 0)),
            pl.BlockSpec((ATOM_F, DIP_F), lambda i: (0, 0)),
            pl.BlockSpec((1, DIP_F), lambda i: (0, 0)),
        ],
        out_specs=pl.BlockSpec((2, R, 32), lambda i: (0, i, 0)),
        out_shape=jax.ShapeDtypeStruct((2, N, 32), jnp.float32),
    )(x, W1, b1.reshape(1, ATOM_F), W2, b2.reshape(1, DIP_F))


_sc_mesh = plsc.VectorSubcoreMesh(core_axis_name="c", subcore_axis_name="s")


@functools.partial(
    pl.kernel,
    out_type=jax.ShapeDtypeStruct((2, N, 3, 32), jnp.float32),
    mesh=_sc_mesh,
    scratch_types=[
        pltpu.VMEM((_NB, 128), jnp.int32),
        pltpu.VMEM((_NB, 128), jnp.int32),
        pltpu.VMEM((3, _B), jnp.float32),
        pltpu.VMEM((_B, 32), jnp.float32),
        pltpu.VMEM((_B, 3, 32), jnp.float32),
        pltpu.VMEM_SHARED((N, 3, 32), jnp.float32),
        pltpu.SemaphoreType.DMA,
    ],
    compiler_params=pltpu.CompilerParams(use_tc_tiling_on_sc=False),
)
def _sc_edge(qh_hbm, src_hbm, dst_hbm, vij_hbm, zeros_hbm, out_hbm,
             sidx, didx, vv, qrows, msg, acc, sem):
    ci = lax.axis_index("c")
    si = lax.axis_index("s")
    row0 = si * _ROWS_PT

    pltpu.sync_copy(zeros_hbm, acc.at[pl.ds(row0, _ROWS_PT)])
    plsc.subcore_barrier()

    off = ci * N

    def chunk_body(it, _):
        grp0 = si * (_EPT // 128) + it * _NB
        ebase = si * _EPT + it * _B
        pltpu.sync_copy(src_hbm.at[pl.ds(grp0, _NB)], sidx)
        pltpu.sync_copy(dst_hbm.at[pl.ds(grp0, _NB)], didx)
        pltpu.sync_copy(vij_hbm.at[:, pl.ds(ebase, _B)], vv)

        for r in range(_NB):
            def adj(k, _):
                sl = pl.ds(k * 16, 16)
                sidx[r, sl] = sidx[r, sl] + off
                return 0
            lax.fori_loop(0, 8, adj, 0)

        descs = [
            pltpu.async_copy(qh_hbm.at[sidx.at[r]],
                             qrows.at[pl.ds(r * 128, 128)], sem)
            for r in range(_NB)
        ]
        for d in descs:
            d.wait()

        def group_body(g, _):
            gsl = pl.ds(g * 16, 16)
            v0g = vv[0, gsl]
            v1g = vv[1, gsl]
            v2g = vv[2, gsl]
            for jj in range(16):
                j = g * 16 + jj
                q0 = qrows[j, pl.ds(0, 16)]
                q1 = qrows[j, pl.ds(16, 16)]
                v0 = v0g[jj]
                v1 = v1g[jj]
                v2 = v2g[jj]
                msg[j, 0, pl.ds(0, 16)] = q0 * v0
                msg[j, 0, pl.ds(16, 16)] = q1 * v0
                msg[j, 1, pl.ds(0, 16)] = q0 * v1
                msg[j, 1, pl.ds(16, 16)] = q1 * v1
                msg[j, 2, pl.ds(0, 16)] = q0 * v2
                msg[j, 2, pl.ds(16, 16)] = q1 * v2
            return 0

        lax.fori_loop(0, _B // 16, group_body, 0)

        for r in range(_NB):
            pltpu.sync_copy(msg.at[pl.ds(r * 128, 128)],
                            acc.at[didx.at[r]], add=True)
        return 0

    lax.fori_loop(0, _NCHUNK, chunk_body, 0)
    plsc.subcore_barrier()

    pltpu.sync_copy(acc.at[pl.ds(row0, _ROWS_PT)],
                    out_hbm.at[ci, pl.ds(row0, _ROWS_PT)])


@jax.jit
def kernel(x, rij, vij, edge_index, W1, b1, W2, b2):
    del rij
    src = edge_index[0].astype(jnp.int32)
    dst = edge_index[1].astype(jnp.int32)
    pad = _EPAD - E
    srcp = jnp.pad(src, (0, pad)).reshape(-1, 128)
    dstp = jnp.pad(dst, (0, pad)).reshape(-1, 128)
    vijp = jnp.pad(vij, ((0, pad), (0, 0))).T

    qh = _mlp(x, W1, b1, W2, b2).reshape(2 * N, 32)
    zeros = jnp.zeros((_ROWS_PT, 3, 32), jnp.float32)
    out = _sc_edge(qh, srcp, dstp, vijp, zeros)
    return out.transpose(1, 0, 3, 2).reshape(N, DIP_F, 3)


# speedup vs baseline: 98.0917x; 1.6544x over previous
import functools

import jax
import jax.numpy as jnp
from jax import lax
from jax.experimental import pallas as pl
from jax.experimental.pallas import tpu as pltpu
from jax.experimental.pallas import tpu_sc as plsc

N = 10000
E = 320000
ATOM_F = 128
DIP_F = 64

_NS = 16
_EPT = 20480
_EPAD = _NS * _EPT
_B = 128
_NB = _B // 128
_NSUP = 4
_SCH = _EPT // _NSUP
_NCHUNK = _SCH // _B
_ROWS_PT = N // _NS


def _mlp_body(x_ref, w1_ref, b1_ref, w2_ref, b2_ref, out_ref):
    h = jnp.dot(x_ref[...], w1_ref[...], preferred_element_type=jnp.float32)
    h = h + b1_ref[...]
    h = h * jax.nn.sigmoid(h)
    q = jnp.dot(h, w2_ref[...], preferred_element_type=jnp.float32)
    q = q + b2_ref[...]
    q = q * jax.nn.sigmoid(q)
    out_ref[0] = q[:, :32]
    out_ref[1] = q[:, 32:]


def _mlp(x, W1, b1, W2, b2):
    R = 1000
    grid = (N // R,)
    return pl.pallas_call(
        _mlp_body,
        grid=grid,
        in_specs=[
            pl.BlockSpec((R, ATOM_F), lambda i: (i, 0)),
            pl.BlockSpec((ATOM_F, ATOM_F), lambda i: (0, 0)),
            pl.BlockSpec((1, ATOM_F), lambda i: (0, 0)),
            pl.BlockSpec((ATOM_F, DIP_F), lambda i: (0, 0)),
            pl.BlockSpec((1, DIP_F), lambda i: (0, 0)),
        ],
        out_specs=pl.BlockSpec((2, R, 32), lambda i: (0, i, 0)),
        out_shape=jax.ShapeDtypeStruct((2, N, 32), jnp.float32),
    )(x, W1, b1.reshape(1, ATOM_F), W2, b2.reshape(1, DIP_F))


_sc_mesh = plsc.VectorSubcoreMesh(core_axis_name="c", subcore_axis_name="s")


@functools.partial(
    pl.kernel,
    out_type=jax.ShapeDtypeStruct((2, N, 3, 32), jnp.float32),
    mesh=_sc_mesh,
    scratch_types=[
        pltpu.VMEM((_SCH // 128, 128), jnp.int32),
        pltpu.VMEM((_SCH // 128, 128), jnp.int32),
        pltpu.VMEM((3, _SCH), jnp.float32),
        pltpu.VMEM((2, _B, 32), jnp.float32),
        pltpu.VMEM((2, _B, 3, 32), jnp.float32),
        pltpu.VMEM_SHARED((N, 3, 32), jnp.float32),
        pltpu.SemaphoreType.DMA,
        pltpu.SemaphoreType.DMA,
    ],
    compiler_params=pltpu.CompilerParams(use_tc_tiling_on_sc=False),
)
def _sc_edge(q0_hbm, q1_hbm, src_hbm, dst_hbm, vij_hbm, zeros_hbm, out_hbm,
             sidx, didx, vv, qrows, msg, acc, semg, sems):
    ci = lax.axis_index("c")
    si = lax.axis_index("s")
    row0 = si * _ROWS_PT

    pltpu.sync_copy(zeros_hbm, acc.at[pl.ds(row0, _ROWS_PT)])
    plsc.subcore_barrier()

    def fire_gather(kk, b):
        for r in range(_NB):
            idx = sidx.at[kk * _NB + r]
            dst = qrows.at[b, pl.ds(r * 128, 128)]

            @pl.when(ci == 0)
            def _():
                pltpu.async_copy(q0_hbm.at[idx], dst, semg)

            @pl.when(ci == 1)
            def _():
                pltpu.async_copy(q1_hbm.at[idx], dst, semg)

    def wait_gather(b):
        pltpu.make_async_copy(q0_hbm.at[pl.ds(0, _B)], qrows.at[b], semg).wait()

    def wait_scatter(b):
        pltpu.make_async_copy(zeros_hbm.at[pl.ds(0, _B)], msg.at[b], sems).wait()

    def compute(kk, b):
        def group_body(g, _):
            gsl = pl.ds(kk * _B + g * 16, 16)
            v0g = vv[0, gsl]
            v1g = vv[1, gsl]
            v2g = vv[2, gsl]
            for jj in range(16):
                j = g * 16 + jj
                q0 = qrows[b, j, pl.ds(0, 16)]
                q1 = qrows[b, j, pl.ds(16, 16)]
                v0 = v0g[jj]
                v1 = v1g[jj]
                v2 = v2g[jj]
                msg[b, j, 0, pl.ds(0, 16)] = q0 * v0
                msg[b, j, 0, pl.ds(16, 16)] = q1 * v0
                msg[b, j, 1, pl.ds(0, 16)] = q0 * v1
                msg[b, j, 1, pl.ds(16, 16)] = q1 * v1
                msg[b, j, 2, pl.ds(0, 16)] = q0 * v2
                msg[b, j, 2, pl.ds(16, 16)] = q1 * v2
            return 0

        lax.fori_loop(0, _B // 16, group_body, 0)

    def fire_scatter(kk, b):
        for r in range(_NB):
            pltpu.async_copy(msg.at[b, pl.ds(r * 128, 128)],
                             acc.at[didx.at[kk * _NB + r]], sems, add=True)

    def super_body(s, _):
        g0 = si * (_EPT // 128) + s * (_SCH // 128)
        e0 = si * _EPT + s * _SCH
        pltpu.sync_copy(src_hbm.at[pl.ds(g0, _SCH // 128)], sidx)
        pltpu.sync_copy(dst_hbm.at[pl.ds(g0, _SCH // 128)], didx)
        pltpu.sync_copy(vij_hbm.at[:, pl.ds(e0, _SCH)], vv)

        fire_gather(0, 0)

        def pair_body(k2, _):
            for b in range(2):
                kk = k2 + b
                if b == 0:
                    fire_gather(kk + 1, 1 - b)
                else:
                    @pl.when(k2 < _NCHUNK - 2)
                    def _():
                        fire_gather(kk + 1, 1 - b)
                wait_gather(b)
                compute(kk, b)
                if b == 0:
                    @pl.when(k2 > 0)
                    def _():
                        wait_scatter(1 - b)
                else:
                    wait_scatter(1 - b)
                fire_scatter(kk, b)
            return 0

        lax.fori_loop(0, _NCHUNK // 2, lambda i, c: pair_body(i * 2, c), 0)
        wait_scatter(1)
        return 0

    lax.fori_loop(0, _NSUP, super_body, 0)
    plsc.subcore_barrier()

    pltpu.sync_copy(acc.at[pl.ds(row0, _ROWS_PT)],
                    out_hbm.at[ci, pl.ds(row0, _ROWS_PT)])


@jax.jit
def kernel(x, rij, vij, edge_index, W1, b1, W2, b2):
    del rij
    src = edge_index[0].astype(jnp.int32)
    dst = edge_index[1].astype(jnp.int32)
    pad = _EPAD - E
    srcp = jnp.pad(src, (0, pad)).reshape(-1, 128)
    dstp = jnp.pad(dst, (0, pad)).reshape(-1, 128)
    vijp = jnp.pad(vij, ((0, pad), (0, 0))).T

    qh = _mlp(x, W1, b1, W2, b2)
    zeros = jnp.zeros((_ROWS_PT, 3, 32), jnp.float32)
    out = _sc_edge(qh[0], qh[1], srcp, dstp, vijp, zeros)
    return out.transpose(1, 0, 3, 2).reshape(N, DIP_F, 3)


# speedup vs baseline: 106.0120x; 1.0807x over previous
import functools

import jax
import jax.numpy as jnp
from jax import lax
from jax.experimental import pallas as pl
from jax.experimental.pallas import tpu as pltpu
from jax.experimental.pallas import tpu_sc as plsc

N = 10000
E = 320000
ATOM_F = 128
DIP_F = 64

_NS = 16
_EPT = 20480
_EPAD = _NS * _EPT
_B = 128
_NB = _B // 128
_NSUP = 4
_SCH = _EPT // _NSUP
_NCHUNK = _SCH // _B
_ROWS_PT = N // _NS


def _mlp_body(x_ref, w1_ref, b1_ref, w2_ref, b2_ref, out_ref):
    h = jnp.dot(x_ref[...], w1_ref[...], preferred_element_type=jnp.float32)
    h = h + b1_ref[...]
    h = h * jax.nn.sigmoid(h)
    q = jnp.dot(h, w2_ref[...], preferred_element_type=jnp.float32)
    q = q + b2_ref[...]
    q = q * jax.nn.sigmoid(q)
    out_ref[0] = q[:, :32]
    out_ref[1] = q[:, 32:]


def _mlp(x, W1, b1, W2, b2):
    R = 1000
    grid = (N // R,)
    return pl.pallas_call(
        _mlp_body,
        grid=grid,
        in_specs=[
            pl.BlockSpec((R, ATOM_F), lambda i: (i, 0)),
            pl.BlockSpec((ATOM_F, ATOM_F), lambda i: (0, 0)),
            pl.BlockSpec((1, ATOM_F), lambda i: (0, 0)),
            pl.BlockSpec((ATOM_F, DIP_F), lambda i: (0, 0)),
            pl.BlockSpec((1, DIP_F), lambda i: (0, 0)),
        ],
        out_specs=pl.BlockSpec((2, R, 32), lambda i: (0, i, 0)),
        out_shape=jax.ShapeDtypeStruct((2, N, 32), jnp.float32),
    )(x, W1, b1.reshape(1, ATOM_F), W2, b2.reshape(1, DIP_F))


_sc_mesh = plsc.VectorSubcoreMesh(core_axis_name="c", subcore_axis_name="s")


@functools.partial(
    pl.kernel,
    out_type=jax.ShapeDtypeStruct((2, N, 3, 32), jnp.float32),
    mesh=_sc_mesh,
    scratch_types=[
        pltpu.VMEM((_SCH // 128, 128), jnp.int32),
        pltpu.VMEM((_SCH // 128, 128), jnp.int32),
        pltpu.VMEM((3, _SCH), jnp.float32),
        pltpu.VMEM((2, _B, 32), jnp.float32),
        pltpu.VMEM((2, _B, 3, 32), jnp.float32),
        pltpu.VMEM_SHARED((N, 3, 32), jnp.float32),
        pltpu.SemaphoreType.DMA,
        pltpu.SemaphoreType.DMA,
    ],
    compiler_params=pltpu.CompilerParams(use_tc_tiling_on_sc=False),
)
def _sc_edge(q0_hbm, q1_hbm, src_hbm, dst_hbm, vij_hbm, zeros_hbm, out_hbm,
             sidx, didx, vv, qrows, msg, acc, semg, sems):
    ci = lax.axis_index("c")
    si = lax.axis_index("s")
    row0 = si * _ROWS_PT

    pltpu.sync_copy(zeros_hbm, acc.at[pl.ds(row0, _ROWS_PT)])
    plsc.subcore_barrier()

    def fire_gather(kk, b):
        for r in range(_NB):
            idx = sidx.at[kk * _NB + r]
            dst = qrows.at[b, pl.ds(r * 128, 128)]

            @pl.when(ci == 0)
            def _():
                pltpu.async_copy(q0_hbm.at[idx], dst, semg)

            @pl.when(ci == 1)
            def _():
                pltpu.async_copy(q1_hbm.at[idx], dst, semg)

    def wait_gather(b):
        pltpu.make_async_copy(q0_hbm.at[pl.ds(0, _B)], qrows.at[b], semg).wait()

    def wait_scatter(b):
        pltpu.make_async_copy(zeros_hbm.at[pl.ds(0, _B)], msg.at[b], sems).wait()

    def compute(kk, b):
        @plsc.parallel_loop(0, _B // 16, unroll=2)
        def group_body(g):
            gsl = pl.ds(kk * _B + g * 16, 16)
            v0g = vv[0, gsl]
            v1g = vv[1, gsl]
            v2g = vv[2, gsl]
            for jj in range(16):
                j = g * 16 + jj
                q0 = qrows[b, j, pl.ds(0, 16)]
                q1 = qrows[b, j, pl.ds(16, 16)]
                v0 = v0g[jj]
                v1 = v1g[jj]
                v2 = v2g[jj]
                msg[b, j, 0, pl.ds(0, 16)] = q0 * v0
                msg[b, j, 0, pl.ds(16, 16)] = q1 * v0
                msg[b, j, 1, pl.ds(0, 16)] = q0 * v1
                msg[b, j, 1, pl.ds(16, 16)] = q1 * v1
                msg[b, j, 2, pl.ds(0, 16)] = q0 * v2
                msg[b, j, 2, pl.ds(16, 16)] = q1 * v2

    def fire_scatter(kk, b):
        for r in range(_NB):
            pltpu.async_copy(msg.at[b, pl.ds(r * 128, 128)],
                             acc.at[didx.at[kk * _NB + r]], sems, add=True)

    def super_body(s, _):
        g0 = si * (_EPT // 128) + s * (_SCH // 128)
        e0 = si * _EPT + s * _SCH
        pltpu.sync_copy(src_hbm.at[pl.ds(g0, _SCH // 128)], sidx)
        pltpu.sync_copy(dst_hbm.at[pl.ds(g0, _SCH // 128)], didx)
        pltpu.sync_copy(vij_hbm.at[:, pl.ds(e0, _SCH)], vv)

        fire_gather(0, 0)

        def pair_body(k2, _):
            for b in range(2):
                kk = k2 + b
                if b == 0:
                    fire_gather(kk + 1, 1 - b)
                else:
                    @pl.when(k2 < _NCHUNK - 2)
                    def _():
                        fire_gather(kk + 1, 1 - b)
                wait_gather(b)
                compute(kk, b)
                if b == 0:
                    @pl.when(k2 > 0)
                    def _():
                        wait_scatter(1 - b)
                else:
                    wait_scatter(1 - b)
                fire_scatter(kk, b)
            return 0

        lax.fori_loop(0, _NCHUNK // 2, lambda i, c: pair_body(i * 2, c), 0)
        wait_scatter(1)
        return 0

    lax.fori_loop(0, _NSUP, super_body, 0)
    plsc.subcore_barrier()

    pltpu.sync_copy(acc.at[pl.ds(row0, _ROWS_PT)],
                    out_hbm.at[ci, pl.ds(row0, _ROWS_PT)])


@jax.jit
def kernel(x, rij, vij, edge_index, W1, b1, W2, b2):
    del rij
    src = edge_index[0].astype(jnp.int32)
    dst = edge_index[1].astype(jnp.int32)
    pad = _EPAD - E
    srcp = jnp.pad(src, (0, pad)).reshape(-1, 128)
    dstp = jnp.pad(dst, (0, pad)).reshape(-1, 128)
    vijp = jnp.pad(vij, ((0, pad), (0, 0))).T

    qh = _mlp(x, W1, b1, W2, b2)
    zeros = jnp.zeros((_ROWS_PT, 3, 32), jnp.float32)
    out = _sc_edge(qh[0], qh[1], srcp, dstp, vijp, zeros)
    return out.transpose(1, 0, 3, 2).reshape(N, DIP_F, 3)
